# baseline (speedup 1.0000x reference)
import functools

import jax
import jax.numpy as jnp
from jax.experimental import pallas as pl
from jax.experimental.pallas import tpu as pltpu

_INTERPRET = False

_D = 32
_IMG = 64
_C = 3
_NL = 85
_SORT_W = 88


def _decode_body(zw_ref, w_ref, b_ref, out_ref):
    x = jnp.dot(zw_ref[...], w_ref[...],
                preferred_element_type=jnp.float32,
                precision=jax.lax.Precision.HIGHEST)
    out_ref[...] = jax.nn.sigmoid(x + b_ref[...][None, :])


def _render_body(glyphs_ref, sorted_ref, zwhere_ref, zpres_ref,
                 out_ref, canvas_ref, step_ref, wcount_ref):
    b = pl.program_id(0)
    canvas_ref[...] = jnp.zeros(canvas_ref.shape, jnp.float32)
    step_ref[...] = jnp.full(step_ref.shape, jnp.inf, jnp.float32)
    wcount_ref[0] = 0
    cnt = sorted_ref[b, _NL]

    lane = jax.lax.broadcasted_iota(jnp.int32, (1, 2 * _IMG), 1)
    gxpair = (lane % _IMG).astype(jnp.float32) * (2.0 / (_IMG - 1)) - 1.0
    selx = lane < _IMG
    gy = jax.lax.broadcasted_iota(jnp.int32, (_IMG, 1), 0).astype(jnp.float32) * (2.0 / (_IMG - 1)) - 1.0
    xp = jax.lax.broadcasted_iota(jnp.int32, (_D, 1), 0).astype(jnp.float32)
    klane = jax.lax.broadcasted_iota(jnp.int32, (1, _IMG), 1)
    kmod = (klane % _D).astype(jnp.float32)
    selk = klane < _D
    lmaskA = selx.astype(jnp.float32) * jnp.ones((_D, 1), jnp.float32)
    lmaskB = 1.0 - lmaskA
    leftmask = jnp.broadcast_to(selx, (_IMG, 2 * _IMG))

    def body(k, carry):
        wq = wcount_ref[0]

        @pl.when((k < cnt) & (wq < _IMG * _IMG))
        def _():
            loc = sorted_ref[b, k]
            ja = 2 * loc
            jb = ja + 1
            pa = zpres_ref[b, ja, 0]
            pb = zpres_ref[b, jb, 0]
            offa = jnp.where(pa == 1, 0.0, 1e9)
            offb = jnp.where(pb == 1, 0.0, 1e9)
            cxa = zwhere_ref[b, ja, 0]
            cya = zwhere_ref[b, ja, 1]
            rwa = 1.0 / (zwhere_ref[b, ja, 2] + 1e-5)
            rha = 1.0 / (zwhere_ref[b, ja, 3] + 1e-5)
            cxb = zwhere_ref[b, jb, 0]
            cyb = zwhere_ref[b, jb, 1]
            rwb = 1.0 / (zwhere_ref[b, jb, 2] + 1e-5)
            rhb = 1.0 / (zwhere_ref[b, jb, 3] + 1e-5)

            sxa = ((gxpair - (2.0 * cxa - 1.0)) * rwa + 1.0) * ((_D - 1) / 2.0) + offa
            sxb = ((gxpair - (2.0 * cxb - 1.0)) * rwb + 1.0) * ((_D - 1) / 2.0) + offb
            sx2 = jnp.where(selx, sxa, sxb)
            rxt2 = jnp.maximum(0.0, 1.0 - jnp.abs(sx2 - xp))

            sya = ((gy - (2.0 * cya - 1.0)) * rha + 1.0) * ((_D - 1) / 2.0) + offa
            syb = ((gy - (2.0 * cyb - 1.0)) * rhb + 1.0) * ((_D - 1) / 2.0) + offb
            syk = jnp.where(selk, sya, syb)
            rycat = jnp.maximum(0.0, 1.0 - jnp.abs(syk - kmod))

            g = glyphs_ref[0, loc]
            a2 = jnp.dot(g, rxt2, preferred_element_type=jnp.float32,
                         precision=jax.lax.Precision.HIGHEST)

            st = step_ref[...]
            r2 = []
            for c in range(_C):
                a2c = a2[c * _D:(c + 1) * _D, :]
                aext = jnp.concatenate([a2c * lmaskA, a2c * lmaskB], axis=0)
                r2.append(jnp.dot(rycat, aext,
                                  preferred_element_type=jnp.float32,
                                  precision=jax.lax.Precision.HIGHEST))
            upd = (r2[0] != 0.0) & (st == jnp.inf)
            for c in range(_C):
                canvas_ref[c] = jnp.where(upd, r2[c], canvas_ref[c])
            step_ref[...] = jnp.where(upd, k.astype(jnp.float32), st)
            wcount_ref[0] = wq + jnp.sum((upd & leftmask).astype(jnp.int32))

        return carry

    jax.lax.fori_loop(0, _NL, body, 0)

    st = step_ref[...]
    use_r = st[:, _IMG:] < st[:, :_IMG]
    for c in range(_C):
        cv = canvas_ref[c]
        out_ref[0, c] = jnp.where(use_r, cv[:, _IMG:], cv[:, :_IMG])


def _sort_standin(z_present, z_depth):
    keys = z_depth[:, :, 0]
    p = z_present[:, :, 0] == 1
    pres2 = p[:, 0::2] | p[:, 1::2]
    sortkey = jnp.where(pres2, -keys, jnp.inf)
    order = jnp.argsort(sortkey, axis=1, stable=True).astype(jnp.int32)
    cnt = jnp.sum(pres2.astype(jnp.int32), axis=1, keepdims=True)
    pad = jnp.zeros((keys.shape[0], _SORT_W - _NL - 1), jnp.int32)
    return jnp.concatenate([order, cnt, pad], axis=1)


def kernel(z_what, z_where, z_present, z_depth, indices, W_dec, b_dec):
    B, NL, Z = z_what.shape

    decoded = pl.pallas_call(
        _decode_body,
        out_shape=jax.ShapeDtypeStruct((B * NL, _C * _D * _D), jnp.float32),
        interpret=_INTERPRET,
    )(z_what.reshape(B * NL, Z), W_dec, b_dec)
    glyphs = decoded.reshape(B, NL, _C * _D, _D)

    sorted_locs = _sort_standin(z_present, z_depth)

    out = pl.pallas_call(
        _render_body,
        grid=(B,),
        in_specs=[
            pl.BlockSpec((1, NL, _C * _D, _D), lambda b: (b, 0, 0, 0)),
            pl.BlockSpec(memory_space=pltpu.SMEM),
            pl.BlockSpec(memory_space=pltpu.SMEM),
            pl.BlockSpec(memory_space=pltpu.SMEM),
        ],
        out_specs=pl.BlockSpec((1, _C, _IMG, _IMG), lambda b: (b, 0, 0, 0)),
        out_shape=jax.ShapeDtypeStruct((B, _C, _IMG, _IMG), jnp.float32),
        scratch_shapes=[
            pltpu.VMEM((_C, _IMG, 2 * _IMG), jnp.float32),
            pltpu.VMEM((_IMG, 2 * _IMG), jnp.float32),
            pltpu.SMEM((1,), jnp.int32),
        ],
        interpret=_INTERPRET,
    )(glyphs, sorted_locs, z_where, z_present)
    return out


# speedup vs baseline: 4.6911x; 4.6911x over previous
import functools

import jax
import jax.numpy as jnp
from jax.experimental import pallas as pl
from jax.experimental.pallas import tpu as pltpu

_INTERPRET = False

_D = 32
_IMG = 64
_C = 3
_NL = 85
_SLOTS = 88
_SORT_W = 96
_CH = 4
_PREC = jax.lax.Precision.DEFAULT


def _decode_body(zw_ref, w_ref, b_ref, out_ref):
    x = jnp.dot(zw_ref[...], w_ref[...],
                preferred_element_type=jnp.float32, precision=_PREC)
    out_ref[...] = jax.nn.sigmoid(x + b_ref[...][None, :])


def _render_body(glyphs_ref, sorted_ref, params_ref,
                 out_ref, canvas_ref, step_ref, done_ref,
                 sxall_ref, syar_ref, sybr_ref):
    b = pl.program_id(0)
    canvas_ref[...] = jnp.zeros(canvas_ref.shape, jnp.float32)
    step_ref[...] = jnp.full(step_ref.shape, jnp.inf, jnp.float32)
    done_ref[0] = 0
    cnt = sorted_ref[b, _SLOTS]

    lane = jax.lax.broadcasted_iota(jnp.int32, (1, 2 * _IMG), 1)
    gxpair = (lane % _IMG).astype(jnp.float32) * (2.0 / (_IMG - 1)) - 1.0
    selx = lane < _IMG
    gyrow = jax.lax.broadcasted_iota(jnp.int32, (1, _IMG), 1).astype(jnp.float32) * (2.0 / (_IMG - 1)) - 1.0
    xp = jax.lax.broadcasted_iota(jnp.int32, (_D, 1), 0).astype(jnp.float32)
    kcol = jax.lax.broadcasted_iota(jnp.int32, (_IMG, 1), 0)
    ksel = kcol < _D
    kmodcol = (kcol % _D).astype(jnp.float32)
    lmaskA = selx.astype(jnp.float32) * jnp.ones((_D, 1), jnp.float32)
    lmaskB = 1.0 - lmaskA
    leftmask = jnp.broadcast_to(selx, (_IMG, 2 * _IMG))
    half = (_D - 1) / 2.0

    pr = params_ref[0]
    cxa, cya, rwa, rha, offa = (pr[:, i:i + 1] for i in range(5))
    cxb, cyb, rwb, rhb, offb = (pr[:, i:i + 1] for i in range(5, 10))
    sx_a = ((gxpair - (2.0 * cxa - 1.0)) * rwa + 1.0) * half + offa
    sx_b = ((gxpair - (2.0 * cxb - 1.0)) * rwb + 1.0) * half + offb
    sxall_ref[...] = jnp.where(selx, sx_a, sx_b)
    syar_ref[...] = ((gyrow - (2.0 * cya - 1.0)) * rha + 1.0) * half + offa
    sybr_ref[...] = ((gyrow - (2.0 * cyb - 1.0)) * rhb + 1.0) * half + offb

    def chunk(ci, carry):
        @pl.when((ci * _CH < cnt) & (done_ref[0] == 0))
        def _():
            st = step_ref[...]
            cv = [canvas_ref[c] for c in range(_C)]
            new_st = st
            for u in range(_CH):
                k = ci * _CH + u
                loc = sorted_ref[b, k]
                sx_row = sxall_ref[pl.ds(k, 1), :]
                rxt2 = jnp.maximum(0.0, 1.0 - jnp.abs(sx_row - xp))
                sya = syar_ref[pl.ds(k, 1), :]
                syb = sybr_ref[pl.ds(k, 1), :]
                sy_sel = jnp.where(ksel, sya, syb)
                rycat_t = jnp.maximum(0.0, 1.0 - jnp.abs(sy_sel - kmodcol))

                g = glyphs_ref[0, loc]
                a2 = jnp.dot(g, rxt2, preferred_element_type=jnp.float32,
                             precision=_PREC)
                r2 = []
                for c in range(_C):
                    a2c = a2[c * _D:(c + 1) * _D, :]
                    aext = jnp.concatenate([a2c * lmaskA, a2c * lmaskB], axis=0)
                    r2.append(jax.lax.dot_general(
                        rycat_t, aext, (((0,), (0,)), ((), ())),
                        preferred_element_type=jnp.float32,
                        precision=_PREC))
                upd = (r2[0] != 0.0) & (new_st == jnp.inf)
                for c in range(_C):
                    cv[c] = jnp.where(upd, r2[c], cv[c])
                new_st = jnp.where(upd, jnp.float32(k), new_st)
            for c in range(_C):
                canvas_ref[c] = cv[c]
            step_ref[...] = new_st
            maxleft = jnp.max(jnp.where(leftmask, new_st, -jnp.inf))
            done_ref[0] = jnp.where(maxleft < jnp.inf, 1, 0)

        return carry

    jax.lax.fori_loop(0, _SLOTS // _CH, chunk, 0)

    st = step_ref[...]
    use_r = st[:, _IMG:] < st[:, :_IMG]
    for c in range(_C):
        cv = canvas_ref[c]
        out_ref[0, c] = jnp.where(use_r, cv[:, _IMG:], cv[:, :_IMG])


def _sort_standin(z_present, z_depth, z_where):
    B = z_depth.shape[0]
    keys = z_depth[:, :, 0]
    p = z_present[:, :, 0] == 1
    pres2 = p[:, 0::2] | p[:, 1::2]
    sortkey = jnp.where(pres2, -keys, jnp.inf)
    order = jnp.argsort(sortkey, axis=1, stable=True).astype(jnp.int32)
    cnt = jnp.sum(pres2.astype(jnp.int32), axis=1, keepdims=True)
    order_p = jnp.concatenate(
        [order, jnp.zeros((B, _SLOTS - _NL), jnp.int32)], axis=1)
    slot_valid = jnp.arange(_SLOTS, dtype=jnp.int32)[None] < cnt
    sorted_full = jnp.concatenate(
        [order_p, cnt, jnp.zeros((B, _SORT_W - _SLOTS - 1), jnp.int32)], axis=1)

    ja = 2 * order_p
    jb = ja + 1

    def gcols(j):
        w = jnp.take_along_axis(z_where, j[:, :, None], axis=1)
        pres = jnp.take_along_axis(p, j, axis=1) & slot_valid
        off = jnp.where(pres, 0.0, 1e9).astype(jnp.float32)
        return (w[:, :, 0], w[:, :, 1],
                1.0 / (w[:, :, 2] + 1e-5), 1.0 / (w[:, :, 3] + 1e-5), off)

    cols = list(gcols(ja)) + list(gcols(jb))
    cols += [jnp.zeros_like(cols[0]), jnp.zeros_like(cols[0])]
    params = jnp.stack(cols, axis=2)
    params = jnp.concatenate(
        [params, jnp.zeros((B, _SORT_W - _SLOTS, 12), jnp.float32)
         .at[:, :, 4].set(1e9).at[:, :, 9].set(1e9)], axis=1)
    return sorted_full, params


def kernel(z_what, z_where, z_present, z_depth, indices, W_dec, b_dec):
    B, NL, Z = z_what.shape

    decoded = pl.pallas_call(
        _decode_body,
        out_shape=jax.ShapeDtypeStruct((B * NL, _C * _D * _D), jnp.float32),
        interpret=_INTERPRET,
    )(z_what.reshape(B * NL, Z), W_dec, b_dec)
    glyphs = decoded.reshape(B, NL, _C * _D, _D)

    sorted_locs, params = _sort_standin(z_present, z_depth, z_where)

    out = pl.pallas_call(
        _render_body,
        grid=(B,),
        in_specs=[
            pl.BlockSpec((1, NL, _C * _D, _D), lambda b: (b, 0, 0, 0)),
            pl.BlockSpec(memory_space=pltpu.SMEM),
            pl.BlockSpec((1, _SORT_W, 12), lambda b: (b, 0, 0)),
        ],
        out_specs=pl.BlockSpec((1, _C, _IMG, _IMG), lambda b: (b, 0, 0, 0)),
        out_shape=jax.ShapeDtypeStruct((B, _C, _IMG, _IMG), jnp.float32),
        scratch_shapes=[
            pltpu.VMEM((_C, _IMG, 2 * _IMG), jnp.float32),
            pltpu.VMEM((_IMG, 2 * _IMG), jnp.float32),
            pltpu.SMEM((1,), jnp.int32),
            pltpu.VMEM((_SORT_W, 2 * _IMG), jnp.float32),
            pltpu.VMEM((_SORT_W, _IMG), jnp.float32),
            pltpu.VMEM((_SORT_W, _IMG), jnp.float32),
        ],
        interpret=_INTERPRET,
    )(glyphs, sorted_locs, params)
    return out
